# baseline (speedup 1.0000x reference)
import functools

import jax
import jax.numpy as jnp
from jax import lax
from jax.experimental import pallas as pl
from jax.experimental.pallas import tpu as pltpu
from jax.experimental.pallas import tpu_sc as plsc

N = 10000
E = 160000
H = 256
NTILES = 32
OWN = 320
HOWN = OWN // 2
NPAD = NTILES * OWN
DUMP = HOWN
NLANE = 16
CAPL = 240
CAP = 2 * NLANE * CAPL
NULLB = CAP
LALLOC = CAP + 64
ECH = 8000
GCH = 48
RING = 4
PCH = 40
PRING = 3
EPT = E // NTILES
NCHUNK = EPT // PCH

_mesh = plsc.VectorSubcoreMesh(core_axis_name="c", subcore_axis_name="s")
_sc_params = pltpu.CompilerParams(needs_layout_passes=False)


def _wid():
    return lax.axis_index("s") * 2 + lax.axis_index("c")


@functools.partial(
    pl.kernel,
    out_type=(
        jax.ShapeDtypeStruct((NTILES, CAP), jnp.int32),
        jax.ShapeDtypeStruct((NTILES, CAP), jnp.int32),
        jax.ShapeDtypeStruct((NTILES, 128), jnp.int32),
        jax.ShapeDtypeStruct((NPAD,), jnp.float32),
        jax.ShapeDtypeStruct((NTILES, CAP), jnp.int32),
        jax.ShapeDtypeStruct((NTILES, CAP), jnp.int32),
        jax.ShapeDtypeStruct((NTILES, 128), jnp.int32),
        jax.ShapeDtypeStruct((NPAD,), jnp.float32),
    ),
    mesh=_mesh,
    compiler_params=_sc_params,
    scratch_types=[
        pltpu.VMEM((ECH,), jnp.int32),
        pltpu.VMEM((ECH,), jnp.int32),
        pltpu.VMEM((ECH,), jnp.int32),
        pltpu.VMEM((ECH,), jnp.int32),
        pltpu.VMEM((CAP,), jnp.int32),
        pltpu.VMEM((CAP,), jnp.int32),
        pltpu.VMEM((CAP,), jnp.int32),
        pltpu.VMEM((CAP,), jnp.int32),
        pltpu.VMEM((128,), jnp.int32),
        pltpu.VMEM((NLANE * (HOWN + 1),), jnp.float32),
        pltpu.VMEM((HOWN,), jnp.float32),
        pltpu.SemaphoreType.DMA,
        pltpu.SemaphoreType.DMA,
        pltpu.SemaphoreType.DMA,
        pltpu.SemaphoreType.DMA,
    ],
)
def _sc_prep(e0_hbm, e1_hbm,
             srev_hbm, drev_hbm, mrev_hbm, irev_hbm,
             smt_hbm, dmt_hbm, mmt_hbm, imt_hbm,
             e0b0, e1b0, e0b1, e1b1,
             srev_v, drev_v, smt_v, dmt_v, mbuf, hist_v, inv_v,
             s00, s10, s01, s11):
    wid = _wid()
    lo = wid * OWN
    lane = lax.iota(jnp.int32, NLANE)
    zi = jnp.zeros(( NLANE,), jnp.int32)
    dumpv = jnp.full((NLANE,), DUMP, jnp.int32)

    def init_lists(k, _):
        sl = pl.ds(k * NLANE, NLANE)
        srev_v[sl] = zi
        drev_v[sl] = dumpv
        smt_v[sl] = zi
        dmt_v[sl] = dumpv
        return 0

    lax.fori_loop(0, CAP // NLANE, init_lists, 0)

    posA0 = lane * CAPL
    posB0 = NLANE * CAPL + lane * CAPL
    limA = posA0 + CAPL
    limB = posB0 + CAPL

    def fire(c, b0, b1, semx, semy):
        off = pl.multiple_of(c * ECH, 8)
        pltpu.async_copy(e0_hbm.at[pl.ds(off, ECH)], b0, semx)
        pltpu.async_copy(e1_hbm.at[pl.ds(off, ECH)], b1, semy)

    def waitpair(c, b0, b1, semx, semy):
        off = pl.multiple_of(c * ECH, 8)
        pltpu.make_async_copy(e0_hbm.at[pl.ds(off, ECH)], b0, semx).wait()
        pltpu.make_async_copy(e1_hbm.at[pl.ds(off, ECH)], b1, semy).wait()

    def scan(b0, b1, pos):
        def vec_body(v, pos):
            pAr, pBr, pAm, pBm = pos
            sl = pl.ds(v * NLANE, NLANE)
            ev0 = b0[sl]
            ev1 = b1[sl]
            dl = ev0 - lo
            inr = (dl >= 0) & (dl < OWN)
            mA = inr & (dl < HOWN) & (pAr < limA)
            mB = inr & (dl >= HOWN) & (pBr < limB)
            plsc.store_scatter(srev_v, [pAr], ev1, mask=mA)
            plsc.store_scatter(drev_v, [pAr], dl, mask=mA)
            plsc.store_scatter(srev_v, [pBr], ev1, mask=mB)
            plsc.store_scatter(drev_v, [pBr], dl - HOWN, mask=mB)
            pAr = pAr + mA.astype(jnp.int32)
            pBr = pBr + mB.astype(jnp.int32)
            dl2 = ev1 - lo
            inr2 = (dl2 >= 0) & (dl2 < OWN)
            mA2 = inr2 & (dl2 < HOWN) & (pAm < limA)
            mB2 = inr2 & (dl2 >= HOWN) & (pBm < limB)
            plsc.store_scatter(smt_v, [pAm], ev0, mask=mA2)
            plsc.store_scatter(dmt_v, [pAm], dl2, mask=mA2)
            plsc.store_scatter(smt_v, [pBm], ev0, mask=mB2)
            plsc.store_scatter(dmt_v, [pBm], dl2 - HOWN, mask=mB2)
            pAm = pAm + mA2.astype(jnp.int32)
            pBm = pBm + mB2.astype(jnp.int32)
            return (pAr, pBr, pAm, pBm)

        return lax.fori_loop(0, ECH // NLANE, vec_body, pos)

    fire(0, e0b0, e1b0, s00, s10)
    fire(1, e0b1, e1b1, s01, s11)

    NCH = E // ECH

    def big_body(g, pos):
        c0 = 2 * g
        waitpair(c0, e0b0, e1b0, s00, s10)
        pos = scan(e0b0, e1b0, pos)

        @pl.when(c0 + 2 < NCH)
        def _():
            fire(c0 + 2, e0b0, e1b0, s00, s10)

        waitpair(c0 + 1, e0b1, e1b1, s01, s11)
        pos = scan(e0b1, e1b1, pos)

        @pl.when(c0 + 3 < NCH)
        def _():
            fire(c0 + 3, e0b1, e1b1, s01, s11)

        return pos

    pos = lax.fori_loop(0, NCH // 2, big_body,
                        (posA0, posB0, posA0, posB0))
    pAr, pBr, pAm, pBm = pos

    def write_counts(pA, pB, mc_hbm):
        for k in range(128 // NLANE):
            if k == 0:
                mbuf[pl.ds(0, NLANE)] = pA - posA0
            elif k == 1:
                mbuf[pl.ds(NLANE, NLANE)] = pB - posB0
            else:
                mbuf[pl.ds(k * NLANE, NLANE)] = zi
        pltpu.sync_copy(mbuf, mc_hbm.at[wid])

    write_counts(pAr, pBr, mrev_hbm)
    write_counts(pAm, pBm, mmt_hbm)
    pltpu.sync_copy(srev_v, srev_hbm.at[wid])
    pltpu.sync_copy(drev_v, drev_hbm.at[wid])
    pltpu.sync_copy(smt_v, smt_hbm.at[wid])
    pltpu.sync_copy(dmt_v, dmt_hbm.at[wid])

    ones = jnp.ones((NLANE,), jnp.float32)
    hstride = lane * (HOWN + 1)
    zf = jnp.zeros((NLANE,), jnp.float32)
    HGRP = NLANE * (HOWN + 1) // NLANE

    def half_hist(dl_v, half, inv_hbm):
        def zero_h(k, _):
            hist_v[pl.ds(k * NLANE, NLANE)] = zf
            return 0

        lax.fori_loop(0, HGRP, zero_h, 0)

        hbase = half * (NLANE * CAPL)

        def hist_body(g, _):
            dv = dl_v[pl.ds(hbase + g * NLANE, NLANE)]
            plsc.addupdate_scatter(hist_v, [hstride + dv], ones)
            return 0

        lax.fori_loop(0, NLANE * CAPL // NLANE, hist_body, 0)

        def inv_body(k, _):
            c16 = jnp.zeros((NLANE,), jnp.float32)
            for l in range(NLANE):
                c16 = c16 + hist_v[pl.ds(l * (HOWN + 1) + k * NLANE, NLANE)]
            inv_v[pl.ds(k * NLANE, NLANE)] = 1.0 / jnp.maximum(c16, 1.0)
            return 0

        lax.fori_loop(0, HOWN // NLANE, inv_body, 0)
        pltpu.sync_copy(inv_v, inv_hbm.at[pl.ds(lo + half * HOWN, HOWN)])

    half_hist(drev_v, 0, irev_hbm)
    half_hist(drev_v, 1, irev_hbm)
    half_hist(dmt_v, 0, imt_hbm)
    half_hist(dmt_v, 1, imt_hbm)


@functools.partial(
    pl.kernel,
    out_type=jax.ShapeDtypeStruct((NPAD, H), jnp.float32),
    mesh=_mesh,
    compiler_params=_sc_params,
    scratch_types=[
        pltpu.VMEM((LALLOC,), jnp.int32),
        pltpu.VMEM((LALLOC,), jnp.int32),
        pltpu.VMEM((128,), jnp.int32),
        pltpu.VMEM((112,), jnp.int32),
        pltpu.VMEM((HOWN + 1, H), jnp.float32),
        pltpu.VMEM((GCH, H), jnp.float32),
        pltpu.VMEM((GCH, H), jnp.float32),
        pltpu.VMEM((GCH, H), jnp.float32),
        pltpu.VMEM((GCH, H), jnp.float32),
        pltpu.SemaphoreType.DMA,
        pltpu.SemaphoreType.DMA,
        pltpu.SemaphoreType.DMA,
        pltpu.SemaphoreType.DMA,
    ],
)
def _sc_agg(p_hbm, slist_hbm, dloc_hbm, mcnt_hbm, out_hbm,
            slist_v, dloc_v, mbuf, btab, acc,
            st0, st1, st2, st3, sem0, sem1, sem2, sem3):
    wid = _wid()
    stages = ((st0, sem0), (st1, sem1), (st2, sem2), (st3, sem3))
    pltpu.sync_copy(mcnt_hbm.at[wid], mbuf)
    pltpu.sync_copy(slist_hbm.at[wid], slist_v.at[pl.ds(0, CAP)])
    pltpu.sync_copy(dloc_hbm.at[wid], dloc_v.at[pl.ds(0, CAP)])

    zi = jnp.zeros((NLANE,), jnp.int32)
    dumpv = jnp.full((NLANE,), DUMP, jnp.int32)
    for k in range((LALLOC - CAP) // NLANE):
        sl = pl.ds(NULLB + k * NLANE, NLANE)
        slist_v[sl] = zi
        dloc_v[sl] = dumpv

    iota = lax.iota(jnp.int32, NLANE)
    zf = jnp.zeros((NLANE,), jnp.float32)
    nullv = jnp.full((NLANE,), NULLB, jnp.int32)

    def fire(slot, stage, sem):
        base = pl.multiple_of(btab[pl.ds(slot, NLANE)][0], 8)
        idx = slist_v.at[pl.ds(base, GCH)]
        pltpu.async_copy(p_hbm.at[idx], stage, sem)

    def proc(slot, stage, sem):
        base = pl.multiple_of(btab[pl.ds(slot, NLANE)][0], 8)
        idx = slist_v.at[pl.ds(base, GCH)]
        pltpu.make_async_copy(p_hbm.at[idx], stage, sem).wait()

        def edge_body(e, _):
            d = dloc_v[pl.ds(base + e, NLANE)][0]
            for j in range(H // NLANE):
                sl = pl.ds(j * NLANE, NLANE)
                plsc.addupdate(acc.at[d, sl], stage[e, sl])
            return 0

        lax.fori_loop(0, GCH, edge_body, 0)

    def phase_body(h, _):
        def zero_row(r, _):
            for j in range(H // NLANE):
                acc[r, pl.ds(j * NLANE, NLANE)] = zf
            return 0

        lax.fori_loop(0, HOWN + 1, zero_row, 0)

        hbase = h * (NLANE * CAPL)

        def build(r, cum):
            mr = mbuf[pl.ds(NLANE * h + r, NLANE)][0]
            trips = (mr + (GCH - 1)) // GCH
            bases = hbase + r * CAPL + GCH * iota
            plsc.store_scatter(btab, [cum + iota], bases, mask=iota < trips)
            return cum + trips

        T = lax.fori_loop(0, NLANE, build, 0)
        plsc.store_scatter(btab, [T + iota], nullv, mask=iota < RING)
        tpad = (T + (RING - 1)) // RING

        @pl.when(T > 0)
        def _():
            for k in range(RING):
                fire(k, *stages[k])

        def ring_body(g, _):
            for k in range(RING):
                slot = RING * g + k
                proc(slot, *stages[k])

                @pl.when(slot + RING < tpad * RING)
                def _():
                    fire(slot + RING, *stages[k])
            return 0

        lax.fori_loop(0, tpad, ring_body, 0)
        pltpu.sync_copy(
            acc.at[pl.ds(0, HOWN)],
            out_hbm.at[pl.ds(wid * OWN + h * HOWN, HOWN)])
        return 0

    lax.fori_loop(0, 2, phase_body, 0)


@functools.partial(
    pl.kernel,
    out_type=jax.ShapeDtypeStruct((E, H), jnp.float32),
    mesh=_mesh,
    compiler_params=_sc_params,
    scratch_types=[
        pltpu.VMEM((EPT,), jnp.int32),
        pltpu.VMEM((EPT,), jnp.int32),
        pltpu.VMEM((PCH, H), jnp.float32),
        pltpu.VMEM((PCH, H), jnp.float32),
        pltpu.VMEM((PCH, H), jnp.float32),
        pltpu.VMEM((PCH, H), jnp.float32),
        pltpu.VMEM((PCH, H), jnp.float32),
        pltpu.VMEM((PCH, H), jnp.float32),
        pltpu.VMEM((PCH, H), jnp.float32),
        pltpu.VMEM((PCH, H), jnp.float32),
        pltpu.VMEM((PCH, H), jnp.float32),
        pltpu.SemaphoreType.DMA,
        pltpu.SemaphoreType.DMA,
        pltpu.SemaphoreType.DMA,
        pltpu.SemaphoreType.DMA,
        pltpu.SemaphoreType.DMA,
        pltpu.SemaphoreType.DMA,
        pltpu.SemaphoreType.DMA,
        pltpu.SemaphoreType.DMA,
        pltpu.SemaphoreType.DMA,
    ],
)
def _sc_pairgather(a_hbm, b_hbm, e0_hbm, e1_hbm, out_hbm,
                   i0_v, i1_v, a0, a1, a2, b0, b1, b2, r0, r1, r2,
                   sa0, sa1, sa2, sb0, sb1, sb2, sw0, sw1, sw2):
    wid = _wid()
    lo = wid * EPT
    pltpu.sync_copy(e0_hbm.at[pl.ds(lo, EPT)], i0_v)
    pltpu.sync_copy(e1_hbm.at[pl.ds(lo, EPT)], i1_v)
    stages = ((a0, b0, r0, sa0, sb0, sw0),
              (a1, b1, r1, sa1, sb1, sw1),
              (a2, b2, r2, sa2, sb2, sw2))

    def fire(c, av, bv, sa, sb):
        off = pl.multiple_of(c * PCH, 8)
        pltpu.async_copy(a_hbm.at[i0_v.at[pl.ds(off, PCH)]], av, sa)
        pltpu.async_copy(b_hbm.at[i1_v.at[pl.ds(off, PCH)]], bv, sb)

    for k in range(PRING):
        fire(k, stages[k][0], stages[k][1], stages[k][3], stages[k][4])

    def proc(c, av, bv, rv, sa, sb, sw):
        off = pl.multiple_of(c * PCH, 8)
        pltpu.make_async_copy(a_hbm.at[i0_v.at[pl.ds(off, PCH)]], av, sa).wait()
        pltpu.make_async_copy(b_hbm.at[i1_v.at[pl.ds(off, PCH)]], bv, sb).wait()

        @pl.when(c >= PRING)
        def _():
            pltpu.make_async_copy(
                rv, out_hbm.at[pl.ds(lo, PCH)], sw).wait()

        def row_body(e, _):
            for j in range(H // NLANE):
                sl = pl.ds(j * NLANE, NLANE)
                rv[e, sl] = av[e, sl] + bv[e, sl]
            return 0

        lax.fori_loop(0, PCH, row_body, 0)
        pltpu.async_copy(rv, out_hbm.at[pl.ds(lo + c * PCH, PCH)], sw)

        @pl.when(c + PRING < NCHUNK)
        def _():
            fire(c + PRING, av, bv, sa, sb)

    def loop_body(c, _):
        m = lax.rem(c, PRING)
        for k in range(PRING):
            @pl.when(m == k)
            def _():
                proc(c, *stages[k])
        return 0

    lax.fori_loop(0, NCHUNK, loop_body, 0)
    for k in range(PRING):
        pltpu.make_async_copy(
            stages[k][2], out_hbm.at[pl.ds(lo, PCH)], stages[k][5]).wait()


_BLK = 1000


def _tc_init_disease(disease_x, lin_W, lin_b, disease_emb):
    def body(dx, w, b, emb, o):
        o[...] = jnp.dot(dx[...], w[...],
                         preferred_element_type=jnp.float32) + b[...] + emb[...]

    return pl.pallas_call(
        body,
        grid=(N // _BLK,),
        in_specs=[
            pl.BlockSpec((_BLK, 10), lambda i: (i, 0)),
            pl.BlockSpec((10, H), lambda i: (0, 0)),
            pl.BlockSpec((1, H), lambda i: (0, 0)),
            pl.BlockSpec((_BLK, H), lambda i: (i, 0)),
        ],
        out_specs=pl.BlockSpec((_BLK, H), lambda i: (i, 0)),
        out_shape=jax.ShapeDtypeStruct((N, H), jnp.float32),
    )(disease_x, lin_W, lin_b, disease_emb)


def _tc_layer_mats(xd, xs, wl_rev, wr_rev, wl_mt, wr_mt):
    def body(xd_r, xs_r, a, b, c, d, p_rev, s_d, p_mt, s_s):
        xdv = xd_r[...]
        xsv = xs_r[...]
        p_rev[...] = jnp.dot(xsv, a[...], preferred_element_type=jnp.float32)
        s_d[...] = jnp.dot(xdv, b[...], preferred_element_type=jnp.float32)
        p_mt[...] = jnp.dot(xdv, c[...], preferred_element_type=jnp.float32)
        s_s[...] = jnp.dot(xsv, d[...], preferred_element_type=jnp.float32)

    full = pl.BlockSpec((H, H), lambda i: (0, 0))
    rows = pl.BlockSpec((_BLK, H), lambda i: (i, 0))
    shp = jax.ShapeDtypeStruct((N, H), jnp.float32)
    return pl.pallas_call(
        body,
        grid=(N // _BLK,),
        in_specs=[rows, rows, full, full, full, full],
        out_specs=[rows, rows, rows, rows],
        out_shape=[shp, shp, shp, shp],
    )(xd, xs, wl_rev, wr_rev, wl_mt, wr_mt)


def _tc_combine(aggd, invd, sd, bld, aggs, invs, ss, bls, relu):
    def body(ad, idv, sdv, bd, as_, isv, ssv, bs, xd_o, xs_o):
        nd = ad[...] * idv[...] + sdv[...] + bd[...]
        ns = as_[...] * isv[...] + ssv[...] + bs[...]
        if relu:
            nd = jnp.maximum(nd, 0.0)
            ns = jnp.maximum(ns, 0.0)
        xd_o[...] = nd
        xs_o[...] = ns

    rows = pl.BlockSpec((_BLK, H), lambda i: (i, 0))
    col = pl.BlockSpec((_BLK, 1), lambda i: (i, 0))
    bias = pl.BlockSpec((1, H), lambda i: (0, 0))
    shp = jax.ShapeDtypeStruct((N, H), jnp.float32)
    return pl.pallas_call(
        body,
        grid=(N // _BLK,),
        in_specs=[rows, col, rows, bias, rows, col, rows, bias],
        out_specs=[rows, rows],
        out_shape=[shp, shp],
    )(aggd, invd, sd, bld, aggs, invs, ss, bls)


def _tc_mlp_head(xd, xs, w_top, w_bot, b1):
    def body(xd_r, xs_r, wt, wb, b, a_o, b_o):
        a_o[...] = jnp.dot(xd_r[...], wt[...],
                           preferred_element_type=jnp.float32)
        b_o[...] = jnp.dot(xs_r[...], wb[...],
                           preferred_element_type=jnp.float32) + b[...]

    rows = pl.BlockSpec((_BLK, H), lambda i: (i, 0))
    full = pl.BlockSpec((H, H), lambda i: (0, 0))
    shp = jax.ShapeDtypeStruct((N, H), jnp.float32)
    return pl.pallas_call(
        body,
        grid=(N // _BLK,),
        in_specs=[rows, rows, full, full, pl.BlockSpec((1, H), lambda i: (0, 0))],
        out_specs=[rows, rows],
        out_shape=[shp, shp],
    )(xd, xs, w_top, w_bot, b1)


_MBLK = 1000


def _tc_mlp(h1, w2, b2, w3, b3, w4, b4):
    def body(h_r, w2r, b2r, w3r, b3r, w4r, b4r, o):
        h = jnp.maximum(h_r[...], 0.0)
        h = jnp.maximum(jnp.dot(h, w2r[...],
                                preferred_element_type=jnp.float32) + b2r[...], 0.0)
        h = jnp.maximum(jnp.dot(h, w3r[...],
                                preferred_element_type=jnp.float32) + b3r[...], 0.0)
        o[...] = jnp.dot(h, w4r[...],
                         preferred_element_type=jnp.float32) + b4r[...]

    return pl.pallas_call(
        body,
        grid=(E // _MBLK,),
        in_specs=[
            pl.BlockSpec((_MBLK, H), lambda i: (i, 0)),
            pl.BlockSpec((H, 128), lambda i: (0, 0)),
            pl.BlockSpec((1, 128), lambda i: (0, 0)),
            pl.BlockSpec((128, 64), lambda i: (0, 0)),
            pl.BlockSpec((1, 64), lambda i: (0, 0)),
            pl.BlockSpec((64, 1), lambda i: (0, 0)),
            pl.BlockSpec((1, 1), lambda i: (0, 0)),
        ],
        out_specs=pl.BlockSpec((_MBLK, 1), lambda i: (i, 0)),
        out_shape=jax.ShapeDtypeStruct((E, 1), jnp.float32),
    )(h1, w2, b2, w3, b3, w4, b4)


def kernel(drug_node_id, disease_x, disease_node_id, edge_index,
           edge_label_index, params):
    xd = params["drug_emb"]
    xs = _tc_init_disease(disease_x, params["lin_W"],
                          params["lin_b"].reshape(1, H), params["disease_emb"])

    (sl_rev, dl_rev, mc_rev, inv_rev,
     sl_mt, dl_mt, mc_mt, inv_mt) = _sc_prep(edge_index[0], edge_index[1])
    invd = inv_rev[:N].reshape(N, 1)
    invs = inv_mt[:N].reshape(N, 1)

    for i in range(4):
        lp = params["convs"][i]
        p_rev, s_d, p_mt, s_s = _tc_layer_mats(
            xd, xs, lp["rev"]["Wl"], lp["rev"]["Wr"],
            lp["mt"]["Wl"], lp["mt"]["Wr"])
        agg_d = _sc_agg(p_rev, sl_rev, dl_rev, mc_rev)
        agg_s = _sc_agg(p_mt, sl_mt, dl_mt, mc_mt)
        xd, xs = _tc_combine(
            agg_d[:N], invd, s_d, lp["rev"]["bl"].reshape(1, H),
            agg_s[:N], invs, s_s, lp["mt"]["bl"].reshape(1, H),
            relu=(i < 3))

    w1, b1 = params["fc"][0]
    a_tab, b_tab = _tc_mlp_head(xd, xs, w1[:H], w1[H:], b1.reshape(1, H))
    h1 = _sc_pairgather(a_tab, b_tab, edge_label_index[0], edge_label_index[1])

    w2, b2 = params["fc"][1]
    w3, b3 = params["fc"][2]
    w4, b4 = params["fc"][3]
    out = _tc_mlp(h1, w2, b2.reshape(1, 128), w3, b3.reshape(1, 64),
                  w4, b4.reshape(1, 1))
    return jnp.squeeze(out, -1)


# speedup vs baseline: 1.5146x; 1.5146x over previous
import functools

import jax
import jax.numpy as jnp
from jax import lax
from jax.experimental import pallas as pl
from jax.experimental.pallas import tpu as pltpu
from jax.experimental.pallas import tpu_sc as plsc

N = 10000
E = 160000
H = 256
NTILES = 32
OWN = 320
HOWN = OWN // 2
NPAD = NTILES * OWN
DUMP = HOWN
NLANE = 16
CAPL = 256
CAP = 2 * NLANE * CAPL
NULLB = CAP
LALLOC = CAP + 64
ECH = 8000
GCH = 32
RING = 4
PCH = 40
PRING = 3
EPT = E // NTILES
NCHUNK = EPT // PCH

_mesh = plsc.VectorSubcoreMesh(core_axis_name="c", subcore_axis_name="s")
_sc_params = pltpu.CompilerParams(needs_layout_passes=False)


def _wid():
    return lax.axis_index("s") * 2 + lax.axis_index("c")


@functools.partial(
    pl.kernel,
    out_type=(
        jax.ShapeDtypeStruct((NTILES, CAP), jnp.int32),
        jax.ShapeDtypeStruct((NTILES, CAP), jnp.int32),
        jax.ShapeDtypeStruct((NTILES, 128), jnp.int32),
        jax.ShapeDtypeStruct((NPAD,), jnp.float32),
        jax.ShapeDtypeStruct((NTILES, CAP), jnp.int32),
        jax.ShapeDtypeStruct((NTILES, CAP), jnp.int32),
        jax.ShapeDtypeStruct((NTILES, 128), jnp.int32),
        jax.ShapeDtypeStruct((NPAD,), jnp.float32),
    ),
    mesh=_mesh,
    compiler_params=_sc_params,
    scratch_types=[
        pltpu.VMEM((ECH,), jnp.int32),
        pltpu.VMEM((ECH,), jnp.int32),
        pltpu.VMEM((ECH,), jnp.int32),
        pltpu.VMEM((ECH,), jnp.int32),
        pltpu.VMEM((CAP,), jnp.int32),
        pltpu.VMEM((CAP,), jnp.int32),
        pltpu.VMEM((CAP,), jnp.int32),
        pltpu.VMEM((CAP,), jnp.int32),
        pltpu.VMEM((128,), jnp.int32),
        pltpu.VMEM((NLANE * (HOWN + 1),), jnp.float32),
        pltpu.VMEM((HOWN,), jnp.float32),
        pltpu.SemaphoreType.DMA,
        pltpu.SemaphoreType.DMA,
        pltpu.SemaphoreType.DMA,
        pltpu.SemaphoreType.DMA,
    ],
)
def _sc_prep(e0_hbm, e1_hbm,
             srev_hbm, drev_hbm, mrev_hbm, irev_hbm,
             smt_hbm, dmt_hbm, mmt_hbm, imt_hbm,
             e0b0, e1b0, e0b1, e1b1,
             srev_v, drev_v, smt_v, dmt_v, mbuf, hist_v, inv_v,
             s00, s10, s01, s11):
    wid = _wid()
    lo = wid * OWN
    lane = lax.iota(jnp.int32, NLANE)
    zi = jnp.zeros(( NLANE,), jnp.int32)
    dumpv = jnp.full((NLANE,), DUMP, jnp.int32)

    def init_lists(k, _):
        sl = pl.ds(k * NLANE, NLANE)
        srev_v[sl] = zi
        drev_v[sl] = dumpv
        smt_v[sl] = zi
        dmt_v[sl] = dumpv
        return 0

    lax.fori_loop(0, CAP // NLANE, init_lists, 0)

    posA0 = lane * CAPL
    posB0 = NLANE * CAPL + lane * CAPL
    limA = posA0 + CAPL
    limB = posB0 + CAPL

    def fire(c, b0, b1, semx, semy):
        off = pl.multiple_of(c * ECH, 8)
        pltpu.async_copy(e0_hbm.at[pl.ds(off, ECH)], b0, semx)
        pltpu.async_copy(e1_hbm.at[pl.ds(off, ECH)], b1, semy)

    def waitpair(c, b0, b1, semx, semy):
        off = pl.multiple_of(c * ECH, 8)
        pltpu.make_async_copy(e0_hbm.at[pl.ds(off, ECH)], b0, semx).wait()
        pltpu.make_async_copy(e1_hbm.at[pl.ds(off, ECH)], b1, semy).wait()

    def scan(b0, b1, pos):
        def vec_body(v, pos):
            pAr, pBr, pAm, pBm = pos
            sl = pl.ds(v * NLANE, NLANE)
            ev0 = b0[sl]
            ev1 = b1[sl]
            dl = ev0 - lo
            inr = (dl >= 0) & (dl < OWN)
            mA = inr & (dl < HOWN) & (pAr < limA)
            mB = inr & (dl >= HOWN) & (pBr < limB)
            plsc.store_scatter(srev_v, [pAr], ev1, mask=mA)
            plsc.store_scatter(drev_v, [pAr], dl, mask=mA)
            plsc.store_scatter(srev_v, [pBr], ev1, mask=mB)
            plsc.store_scatter(drev_v, [pBr], dl - HOWN, mask=mB)
            pAr = pAr + mA.astype(jnp.int32)
            pBr = pBr + mB.astype(jnp.int32)
            dl2 = ev1 - lo
            inr2 = (dl2 >= 0) & (dl2 < OWN)
            mA2 = inr2 & (dl2 < HOWN) & (pAm < limA)
            mB2 = inr2 & (dl2 >= HOWN) & (pBm < limB)
            plsc.store_scatter(smt_v, [pAm], ev0, mask=mA2)
            plsc.store_scatter(dmt_v, [pAm], dl2, mask=mA2)
            plsc.store_scatter(smt_v, [pBm], ev0, mask=mB2)
            plsc.store_scatter(dmt_v, [pBm], dl2 - HOWN, mask=mB2)
            pAm = pAm + mA2.astype(jnp.int32)
            pBm = pBm + mB2.astype(jnp.int32)
            return (pAr, pBr, pAm, pBm)

        return lax.fori_loop(0, ECH // NLANE, vec_body, pos)

    fire(0, e0b0, e1b0, s00, s10)
    fire(1, e0b1, e1b1, s01, s11)

    NCH = E // ECH

    def big_body(g, pos):
        c0 = 2 * g
        waitpair(c0, e0b0, e1b0, s00, s10)
        pos = scan(e0b0, e1b0, pos)

        @pl.when(c0 + 2 < NCH)
        def _():
            fire(c0 + 2, e0b0, e1b0, s00, s10)

        waitpair(c0 + 1, e0b1, e1b1, s01, s11)
        pos = scan(e0b1, e1b1, pos)

        @pl.when(c0 + 3 < NCH)
        def _():
            fire(c0 + 3, e0b1, e1b1, s01, s11)

        return pos

    pos = lax.fori_loop(0, NCH // 2, big_body,
                        (posA0, posB0, posA0, posB0))
    pAr, pBr, pAm, pBm = pos

    def write_counts(pA, pB, mc_hbm):
        for k in range(128 // NLANE):
            if k == 0:
                mbuf[pl.ds(0, NLANE)] = pA - posA0
            elif k == 1:
                mbuf[pl.ds(NLANE, NLANE)] = pB - posB0
            else:
                mbuf[pl.ds(k * NLANE, NLANE)] = zi
        pltpu.sync_copy(mbuf, mc_hbm.at[wid])

    write_counts(pAr, pBr, mrev_hbm)
    write_counts(pAm, pBm, mmt_hbm)
    pltpu.sync_copy(srev_v, srev_hbm.at[wid])
    pltpu.sync_copy(drev_v, drev_hbm.at[wid])
    pltpu.sync_copy(smt_v, smt_hbm.at[wid])
    pltpu.sync_copy(dmt_v, dmt_hbm.at[wid])

    ones = jnp.ones((NLANE,), jnp.float32)
    hstride = lane * (HOWN + 1)
    zf = jnp.zeros((NLANE,), jnp.float32)
    HGRP = NLANE * (HOWN + 1) // NLANE

    def half_hist(dl_v, half, inv_hbm):
        def zero_h(k, _):
            hist_v[pl.ds(k * NLANE, NLANE)] = zf
            return 0

        lax.fori_loop(0, HGRP, zero_h, 0)

        hbase = half * (NLANE * CAPL)

        def hist_body(g, _):
            dv = dl_v[pl.ds(hbase + g * NLANE, NLANE)]
            plsc.addupdate_scatter(hist_v, [hstride + dv], ones)
            return 0

        lax.fori_loop(0, NLANE * CAPL // NLANE, hist_body, 0)

        def inv_body(k, _):
            c16 = jnp.zeros((NLANE,), jnp.float32)
            for l in range(NLANE):
                c16 = c16 + hist_v[pl.ds(l * (HOWN + 1) + k * NLANE, NLANE)]
            inv_v[pl.ds(k * NLANE, NLANE)] = 1.0 / jnp.maximum(c16, 1.0)
            return 0

        lax.fori_loop(0, HOWN // NLANE, inv_body, 0)
        pltpu.sync_copy(inv_v, inv_hbm.at[pl.ds(lo + half * HOWN, HOWN)])

    half_hist(drev_v, 0, irev_hbm)
    half_hist(drev_v, 1, irev_hbm)
    half_hist(dmt_v, 0, imt_hbm)
    half_hist(dmt_v, 1, imt_hbm)


@functools.partial(
    pl.kernel,
    out_type=jax.ShapeDtypeStruct((NPAD, H), jnp.float32),
    mesh=_mesh,
    compiler_params=_sc_params,
    scratch_types=[
        pltpu.VMEM((LALLOC,), jnp.int32),
        pltpu.VMEM((LALLOC,), jnp.int32),
        pltpu.VMEM((128,), jnp.int32),
        pltpu.VMEM((160,), jnp.int32),
        pltpu.VMEM((HOWN + 1, H), jnp.float32),
        pltpu.VMEM((GCH, H), jnp.float32),
        pltpu.VMEM((GCH, H), jnp.float32),
        pltpu.VMEM((GCH, H), jnp.float32),
        pltpu.VMEM((GCH, H), jnp.float32),
        pltpu.SemaphoreType.DMA,
        pltpu.SemaphoreType.DMA,
        pltpu.SemaphoreType.DMA,
        pltpu.SemaphoreType.DMA,
    ],
)
def _sc_agg(p_hbm, slist_hbm, dloc_hbm, mcnt_hbm, out_hbm,
            slist_v, dloc_v, mbuf, btab, acc,
            st0, st1, st2, st3, sem0, sem1, sem2, sem3):
    wid = _wid()
    stages = ((st0, sem0), (st1, sem1), (st2, sem2), (st3, sem3))
    pltpu.sync_copy(mcnt_hbm.at[wid], mbuf)
    pltpu.sync_copy(slist_hbm.at[wid], slist_v.at[pl.ds(0, CAP)])
    pltpu.sync_copy(dloc_hbm.at[wid], dloc_v.at[pl.ds(0, CAP)])

    zi = jnp.zeros((NLANE,), jnp.int32)
    dumpv = jnp.full((NLANE,), DUMP, jnp.int32)
    for k in range((LALLOC - CAP) // NLANE):
        sl = pl.ds(NULLB + k * NLANE, NLANE)
        slist_v[sl] = zi
        dloc_v[sl] = dumpv

    iota = lax.iota(jnp.int32, NLANE)
    zf = jnp.zeros((NLANE,), jnp.float32)
    nullv = jnp.full((NLANE,), NULLB, jnp.int32)

    def fire(slot, stage, sem):
        base = pl.multiple_of(btab[pl.ds(slot, NLANE)][0], 8)
        idx = slist_v.at[pl.ds(base, GCH)]
        pltpu.async_copy(p_hbm.at[idx], stage, sem)

    def proc(slot, stage, sem):
        base = pl.multiple_of(btab[pl.ds(slot, NLANE)][0], 8)
        idx = slist_v.at[pl.ds(base, GCH)]
        pltpu.make_async_copy(p_hbm.at[idx], stage, sem).wait()

        for eg in range(GCH // NLANE):
            dv = dloc_v[pl.ds(base + eg * NLANE, NLANE)]
            for el in range(NLANE):
                d = dv[el]
                e = eg * NLANE + el
                for j in range(H // NLANE):
                    sl = pl.ds(j * NLANE, NLANE)
                    plsc.addupdate(acc.at[d, sl], stage[e, sl])

    def phase_body(h, _):
        def zero_row(r, _):
            for j in range(H // NLANE):
                acc[r, pl.ds(j * NLANE, NLANE)] = zf
            return 0

        lax.fori_loop(0, HOWN + 1, zero_row, 0)

        hbase = h * (NLANE * CAPL)

        def build(r, cum):
            mr = mbuf[pl.ds(NLANE * h + r, NLANE)][0]
            trips = (mr + (GCH - 1)) // GCH
            bases = hbase + r * CAPL + GCH * iota
            plsc.store_scatter(btab, [cum + iota], bases, mask=iota < trips)
            return cum + trips

        T = lax.fori_loop(0, NLANE, build, 0)
        plsc.store_scatter(btab, [T + iota], nullv, mask=iota < RING)
        tpad = (T + (RING - 1)) // RING

        @pl.when(T > 0)
        def _():
            for k in range(RING):
                fire(k, *stages[k])

        def ring_body(g, _):
            for k in range(RING):
                slot = RING * g + k
                proc(slot, *stages[k])

                @pl.when(slot + RING < tpad * RING)
                def _():
                    fire(slot + RING, *stages[k])
            return 0

        lax.fori_loop(0, tpad, ring_body, 0)
        pltpu.sync_copy(
            acc.at[pl.ds(0, HOWN)],
            out_hbm.at[pl.ds(wid * OWN + h * HOWN, HOWN)])
        return 0

    lax.fori_loop(0, 2, phase_body, 0)


@functools.partial(
    pl.kernel,
    out_type=jax.ShapeDtypeStruct((E, H), jnp.float32),
    mesh=_mesh,
    compiler_params=_sc_params,
    scratch_types=[
        pltpu.VMEM((EPT,), jnp.int32),
        pltpu.VMEM((EPT,), jnp.int32),
        pltpu.VMEM((PCH, H), jnp.float32),
        pltpu.VMEM((PCH, H), jnp.float32),
        pltpu.VMEM((PCH, H), jnp.float32),
        pltpu.VMEM((PCH, H), jnp.float32),
        pltpu.VMEM((PCH, H), jnp.float32),
        pltpu.VMEM((PCH, H), jnp.float32),
        pltpu.VMEM((PCH, H), jnp.float32),
        pltpu.VMEM((PCH, H), jnp.float32),
        pltpu.VMEM((PCH, H), jnp.float32),
        pltpu.SemaphoreType.DMA,
        pltpu.SemaphoreType.DMA,
        pltpu.SemaphoreType.DMA,
        pltpu.SemaphoreType.DMA,
        pltpu.SemaphoreType.DMA,
        pltpu.SemaphoreType.DMA,
        pltpu.SemaphoreType.DMA,
        pltpu.SemaphoreType.DMA,
        pltpu.SemaphoreType.DMA,
    ],
)
def _sc_pairgather(a_hbm, b_hbm, e0_hbm, e1_hbm, out_hbm,
                   i0_v, i1_v, a0, a1, a2, b0, b1, b2, r0, r1, r2,
                   sa0, sa1, sa2, sb0, sb1, sb2, sw0, sw1, sw2):
    wid = _wid()
    lo = wid * EPT
    pltpu.sync_copy(e0_hbm.at[pl.ds(lo, EPT)], i0_v)
    pltpu.sync_copy(e1_hbm.at[pl.ds(lo, EPT)], i1_v)
    stages = ((a0, b0, r0, sa0, sb0, sw0),
              (a1, b1, r1, sa1, sb1, sw1),
              (a2, b2, r2, sa2, sb2, sw2))

    def fire(c, av, bv, sa, sb):
        off = pl.multiple_of(c * PCH, 8)
        pltpu.async_copy(a_hbm.at[i0_v.at[pl.ds(off, PCH)]], av, sa)
        pltpu.async_copy(b_hbm.at[i1_v.at[pl.ds(off, PCH)]], bv, sb)

    for k in range(PRING):
        fire(k, stages[k][0], stages[k][1], stages[k][3], stages[k][4])

    def proc(c, av, bv, rv, sa, sb, sw):
        off = pl.multiple_of(c * PCH, 8)
        pltpu.make_async_copy(a_hbm.at[i0_v.at[pl.ds(off, PCH)]], av, sa).wait()
        pltpu.make_async_copy(b_hbm.at[i1_v.at[pl.ds(off, PCH)]], bv, sb).wait()

        @pl.when(c >= PRING)
        def _():
            pltpu.make_async_copy(
                rv, out_hbm.at[pl.ds(lo, PCH)], sw).wait()

        def row_body(e, _):
            for j in range(H // NLANE):
                sl = pl.ds(j * NLANE, NLANE)
                rv[e, sl] = av[e, sl] + bv[e, sl]
            return 0

        lax.fori_loop(0, PCH, row_body, 0)
        pltpu.async_copy(rv, out_hbm.at[pl.ds(lo + c * PCH, PCH)], sw)

        @pl.when(c + PRING < NCHUNK)
        def _():
            fire(c + PRING, av, bv, sa, sb)

    def loop_body(c, _):
        m = lax.rem(c, PRING)
        for k in range(PRING):
            @pl.when(m == k)
            def _():
                proc(c, *stages[k])
        return 0

    lax.fori_loop(0, NCHUNK, loop_body, 0)
    for k in range(PRING):
        pltpu.make_async_copy(
            stages[k][2], out_hbm.at[pl.ds(lo, PCH)], stages[k][5]).wait()


_BLK = 1000


def _tc_init_disease(disease_x, lin_W, lin_b, disease_emb):
    def body(dx, w, b, emb, o):
        o[...] = jnp.dot(dx[...], w[...],
                         preferred_element_type=jnp.float32) + b[...] + emb[...]

    return pl.pallas_call(
        body,
        grid=(N // _BLK,),
        in_specs=[
            pl.BlockSpec((_BLK, 10), lambda i: (i, 0)),
            pl.BlockSpec((10, H), lambda i: (0, 0)),
            pl.BlockSpec((1, H), lambda i: (0, 0)),
            pl.BlockSpec((_BLK, H), lambda i: (i, 0)),
        ],
        out_specs=pl.BlockSpec((_BLK, H), lambda i: (i, 0)),
        out_shape=jax.ShapeDtypeStruct((N, H), jnp.float32),
    )(disease_x, lin_W, lin_b, disease_emb)


def _tc_layer_mats(xd, xs, wl_rev, wr_rev, wl_mt, wr_mt):
    def body(xd_r, xs_r, a, b, c, d, p_rev, s_d, p_mt, s_s):
        xdv = xd_r[...]
        xsv = xs_r[...]
        p_rev[...] = jnp.dot(xsv, a[...], preferred_element_type=jnp.float32)
        s_d[...] = jnp.dot(xdv, b[...], preferred_element_type=jnp.float32)
        p_mt[...] = jnp.dot(xdv, c[...], preferred_element_type=jnp.float32)
        s_s[...] = jnp.dot(xsv, d[...], preferred_element_type=jnp.float32)

    full = pl.BlockSpec((H, H), lambda i: (0, 0))
    rows = pl.BlockSpec((_BLK, H), lambda i: (i, 0))
    shp = jax.ShapeDtypeStruct((N, H), jnp.float32)
    return pl.pallas_call(
        body,
        grid=(N // _BLK,),
        in_specs=[rows, rows, full, full, full, full],
        out_specs=[rows, rows, rows, rows],
        out_shape=[shp, shp, shp, shp],
    )(xd, xs, wl_rev, wr_rev, wl_mt, wr_mt)


def _tc_combine(aggd, invd, sd, bld, aggs, invs, ss, bls, relu):
    def body(ad, idv, sdv, bd, as_, isv, ssv, bs, xd_o, xs_o):
        nd = ad[...] * idv[...] + sdv[...] + bd[...]
        ns = as_[...] * isv[...] + ssv[...] + bs[...]
        if relu:
            nd = jnp.maximum(nd, 0.0)
            ns = jnp.maximum(ns, 0.0)
        xd_o[...] = nd
        xs_o[...] = ns

    rows = pl.BlockSpec((_BLK, H), lambda i: (i, 0))
    col = pl.BlockSpec((_BLK, 1), lambda i: (i, 0))
    bias = pl.BlockSpec((1, H), lambda i: (0, 0))
    shp = jax.ShapeDtypeStruct((N, H), jnp.float32)
    return pl.pallas_call(
        body,
        grid=(N // _BLK,),
        in_specs=[rows, col, rows, bias, rows, col, rows, bias],
        out_specs=[rows, rows],
        out_shape=[shp, shp],
    )(aggd, invd, sd, bld, aggs, invs, ss, bls)


def _tc_mlp_head(xd, xs, w_top, w_bot, b1):
    def body(xd_r, xs_r, wt, wb, b, a_o, b_o):
        a_o[...] = jnp.dot(xd_r[...], wt[...],
                           preferred_element_type=jnp.float32)
        b_o[...] = jnp.dot(xs_r[...], wb[...],
                           preferred_element_type=jnp.float32) + b[...]

    rows = pl.BlockSpec((_BLK, H), lambda i: (i, 0))
    full = pl.BlockSpec((H, H), lambda i: (0, 0))
    shp = jax.ShapeDtypeStruct((N, H), jnp.float32)
    return pl.pallas_call(
        body,
        grid=(N // _BLK,),
        in_specs=[rows, rows, full, full, pl.BlockSpec((1, H), lambda i: (0, 0))],
        out_specs=[rows, rows],
        out_shape=[shp, shp],
    )(xd, xs, w_top, w_bot, b1)


_MBLK = 1000


def _tc_mlp(h1, w2, b2, w3, b3, w4, b4):
    def body(h_r, w2r, b2r, w3r, b3r, w4r, b4r, o):
        h = jnp.maximum(h_r[...], 0.0)
        h = jnp.maximum(jnp.dot(h, w2r[...],
                                preferred_element_type=jnp.float32) + b2r[...], 0.0)
        h = jnp.maximum(jnp.dot(h, w3r[...],
                                preferred_element_type=jnp.float32) + b3r[...], 0.0)
        o[...] = jnp.dot(h, w4r[...],
                         preferred_element_type=jnp.float32) + b4r[...]

    return pl.pallas_call(
        body,
        grid=(E // _MBLK,),
        in_specs=[
            pl.BlockSpec((_MBLK, H), lambda i: (i, 0)),
            pl.BlockSpec((H, 128), lambda i: (0, 0)),
            pl.BlockSpec((1, 128), lambda i: (0, 0)),
            pl.BlockSpec((128, 64), lambda i: (0, 0)),
            pl.BlockSpec((1, 64), lambda i: (0, 0)),
            pl.BlockSpec((64, 1), lambda i: (0, 0)),
            pl.BlockSpec((1, 1), lambda i: (0, 0)),
        ],
        out_specs=pl.BlockSpec((_MBLK, 1), lambda i: (i, 0)),
        out_shape=jax.ShapeDtypeStruct((E, 1), jnp.float32),
    )(h1, w2, b2, w3, b3, w4, b4)


def kernel(drug_node_id, disease_x, disease_node_id, edge_index,
           edge_label_index, params):
    xd = params["drug_emb"]
    xs = _tc_init_disease(disease_x, params["lin_W"],
                          params["lin_b"].reshape(1, H), params["disease_emb"])

    (sl_rev, dl_rev, mc_rev, inv_rev,
     sl_mt, dl_mt, mc_mt, inv_mt) = _sc_prep(edge_index[0], edge_index[1])
    invd = inv_rev[:N].reshape(N, 1)
    invs = inv_mt[:N].reshape(N, 1)

    for i in range(4):
        lp = params["convs"][i]
        p_rev, s_d, p_mt, s_s = _tc_layer_mats(
            xd, xs, lp["rev"]["Wl"], lp["rev"]["Wr"],
            lp["mt"]["Wl"], lp["mt"]["Wr"])
        agg_d = _sc_agg(p_rev, sl_rev, dl_rev, mc_rev)
        agg_s = _sc_agg(p_mt, sl_mt, dl_mt, mc_mt)
        xd, xs = _tc_combine(
            agg_d[:N], invd, s_d, lp["rev"]["bl"].reshape(1, H),
            agg_s[:N], invs, s_s, lp["mt"]["bl"].reshape(1, H),
            relu=(i < 3))

    w1, b1 = params["fc"][0]
    a_tab, b_tab = _tc_mlp_head(xd, xs, w1[:H], w1[H:], b1.reshape(1, H))
    h1 = _sc_pairgather(a_tab, b_tab, edge_label_index[0], edge_label_index[1])

    w2, b2 = params["fc"][1]
    w3, b3 = params["fc"][2]
    w4, b4 = params["fc"][3]
    out = _tc_mlp(h1, w2, b2.reshape(1, 128), w3, b3.reshape(1, 64),
                  w4, b4.reshape(1, 1))
    return jnp.squeeze(out, -1)


# speedup vs baseline: 1.7104x; 1.1293x over previous
import functools

import jax
import jax.numpy as jnp
from jax import lax
from jax.experimental import pallas as pl
from jax.experimental.pallas import tpu as pltpu
from jax.experimental.pallas import tpu_sc as plsc

N = 10000
E = 160000
H = 256
NTILES = 32
OWN = 320
HOWN = OWN // 2
NPAD = NTILES * OWN
DUMP = HOWN
NLANE = 16
CAPL = 144
NGRP = 4
CAP = NGRP * NLANE * CAPL
SQN = 2560
SH = 2 * SQN
CAPL2 = 80
NGRP2 = 8
CAP2 = NGRP2 * NLANE * CAPL2
NULLB = CAP2
LALLOC = CAP2 + 96
SSTG = SQN // NLANE
ECH = 8000
GCH = 80
RING = 2
PCH = 40
PRING = 3
EPT = E // NTILES
NCHUNK = EPT // PCH

_mesh = plsc.VectorSubcoreMesh(core_axis_name="c", subcore_axis_name="s")
_sc_params = pltpu.CompilerParams(needs_layout_passes=False)


def _wid():
    return lax.axis_index("s") * 2 + lax.axis_index("c")


@functools.partial(
    pl.kernel,
    out_type=(
        jax.ShapeDtypeStruct((NTILES, CAP2), jnp.int32),
        jax.ShapeDtypeStruct((NTILES, CAP2), jnp.int32),
        jax.ShapeDtypeStruct((NTILES, 128), jnp.int32),
        jax.ShapeDtypeStruct((NPAD,), jnp.float32),
        jax.ShapeDtypeStruct((NTILES, CAP2), jnp.int32),
        jax.ShapeDtypeStruct((NTILES, CAP2), jnp.int32),
        jax.ShapeDtypeStruct((NTILES, 128), jnp.int32),
        jax.ShapeDtypeStruct((NPAD,), jnp.float32),
    ),
    mesh=_mesh,
    compiler_params=_sc_params,
    scratch_types=[
        pltpu.VMEM((ECH,), jnp.int32),
        pltpu.VMEM((ECH,), jnp.int32),
        pltpu.VMEM((ECH,), jnp.int32),
        pltpu.VMEM((ECH,), jnp.int32),
        pltpu.VMEM((CAP,), jnp.int32),
        pltpu.VMEM((CAP,), jnp.int32),
        pltpu.VMEM((CAP,), jnp.int32),
        pltpu.VMEM((CAP,), jnp.int32),
        pltpu.VMEM((CAP2,), jnp.int32),
        pltpu.VMEM((CAP2,), jnp.int32),
        pltpu.VMEM((80,), jnp.int32),
        pltpu.VMEM((NGRP2 * NLANE,), jnp.int32),
        pltpu.VMEM((128,), jnp.int32),
        pltpu.VMEM((NLANE * (HOWN + 1),), jnp.float32),
        pltpu.VMEM((HOWN,), jnp.float32),
        pltpu.SemaphoreType.DMA,
        pltpu.SemaphoreType.DMA,
        pltpu.SemaphoreType.DMA,
        pltpu.SemaphoreType.DMA,
    ],
)
def _sc_prep(e0_hbm, e1_hbm,
             srev_hbm, drev_hbm, mrev_hbm, irev_hbm,
             smt_hbm, dmt_hbm, mmt_hbm, imt_hbm,
             e0b0, e1b0, e0b1, e1b1,
             srev_v, drev_v, smt_v, dmt_v, s2_v, d2_v, pbuf, posb,
             mbuf, hist_v, inv_v,
             s00, s10, s01, s11):
    wid = _wid()
    lo = wid * OWN
    lane = lax.iota(jnp.int32, NLANE)
    zi = jnp.zeros(( NLANE,), jnp.int32)
    dumpv = jnp.full((NLANE,), DUMP, jnp.int32)

    def init_lists(k, _):
        sl = pl.ds(k * NLANE, NLANE)
        srev_v[sl] = zi
        drev_v[sl] = dumpv
        smt_v[sl] = zi
        dmt_v[sl] = dumpv
        return 0

    lax.fori_loop(0, CAP // NLANE, init_lists, 0)

    pos0 = [(g * NLANE + lane) * CAPL for g in range(NGRP)]
    lims = [p + CAPL for p in pos0]

    def fire(c, b0, b1, semx, semy):
        off = pl.multiple_of(c * ECH, 8)
        pltpu.async_copy(e0_hbm.at[pl.ds(off, ECH)], b0, semx)
        pltpu.async_copy(e1_hbm.at[pl.ds(off, ECH)], b1, semy)

    def waitpair(c, b0, b1, semx, semy):
        off = pl.multiple_of(c * ECH, 8)
        pltpu.make_async_copy(e0_hbm.at[pl.ds(off, ECH)], b0, semx).wait()
        pltpu.make_async_copy(e1_hbm.at[pl.ds(off, ECH)], b1, semy).wait()

    def scan(b0, b1, pos):
        def one_dir(ev_d, ev_s, s_ref, d_ref, p4):
            dl = ev_d - lo
            valid = (dl >= 0) & (dl < OWN)
            is_b = dl >= HOWN
            dlh = jnp.where(is_b, dl - HOWN, dl)
            is_s1 = ev_s >= SH
            srl = jnp.where(is_s1, ev_s - SH, ev_s)
            out = []
            for g in range(NGRP):
                hg, sg = g & 1, g >> 1
                mh = is_b if hg else jnp.logical_not(is_b)
                ms = is_s1 if sg else jnp.logical_not(is_s1)
                m = valid & mh & ms & (p4[g] < lims[g])
                plsc.store_scatter(s_ref, [p4[g]], srl, mask=m)
                plsc.store_scatter(d_ref, [p4[g]], dlh, mask=m)
                out.append(p4[g] + m.astype(jnp.int32))
            return out

        def vec_body(v, pos):
            sl = pl.ds(v * NLANE, NLANE)
            ev0 = b0[sl]
            ev1 = b1[sl]
            pr = one_dir(ev0, ev1, srev_v, drev_v, pos[0:4])
            pm = one_dir(ev1, ev0, smt_v, dmt_v, pos[4:8])
            return tuple(pr + pm)

        return lax.fori_loop(0, ECH // NLANE, vec_body, pos)

    fire(0, e0b0, e1b0, s00, s10)
    fire(1, e0b1, e1b1, s01, s11)

    NCH = E // ECH

    def big_body(g, pos):
        c0 = 2 * g
        waitpair(c0, e0b0, e1b0, s00, s10)
        pos = scan(e0b0, e1b0, pos)

        @pl.when(c0 + 2 < NCH)
        def _():
            fire(c0 + 2, e0b0, e1b0, s00, s10)

        waitpair(c0 + 1, e0b1, e1b1, s01, s11)
        pos = scan(e0b1, e1b1, pos)

        @pl.when(c0 + 3 < NCH)
        def _():
            fire(c0 + 3, e0b1, e1b1, s01, s11)

        return pos

    pos = lax.fori_loop(0, NCH // 2, big_body,
                        tuple(pos0 + pos0))
    iota = lane

    pos20 = [(g * NLANE + lane) * CAPL2 for g in range(NGRP2)]

    def passb(s_v, d_v, p4, mc_hbm, s2_hbm, d2_hbm):
        def initb(k, _):
            sl = pl.ds(k * NLANE, NLANE)
            s2_v[sl] = zi
            d2_v[sl] = dumpv
            return 0

        lax.fori_loop(0, CAP2 // NLANE, initb, 0)
        for g in range(NGRP):
            pbuf[pl.ds(g * NLANE, NLANE)] = p4[g] - pos0[g]
        for g in range(NGRP2):
            posb[pl.ds(g * NLANE, NLANE)] = pos20[g]

        def region_body(ridx, _):
            g = ridx // NLANE
            sh = g // 2
            h = g - 2 * sh
            mr = pbuf[pl.ds(ridx, NLANE)][0]
            base = ridx * CAPL
            nv = (mr + (NLANE - 1)) // NLANE

            def vec_body(v, _):
                off = base + v * NLANE
                srl = s_v[pl.ds(off, NLANE)]
                dl = d_v[pl.ds(off, NLANE)]
                valid = iota < (mr - v * NLANE)
                q = srl >= SQN
                srl2 = jnp.where(q, srl - SQN, srl)
                for qq in range(2):
                    g2 = (sh * 2 + qq) * 2 + h
                    pv = posb[pl.ds(g2 * NLANE, NLANE)]
                    mq = q if qq else jnp.logical_not(q)
                    limv = (g2 * NLANE + iota + 1) * CAPL2
                    m = valid & mq & (pv < limv)
                    plsc.store_scatter(s2_v, [pv], srl2, mask=m)
                    plsc.store_scatter(d2_v, [pv], dl, mask=m)
                    posb[pl.ds(g2 * NLANE, NLANE)] = pv + m.astype(jnp.int32)
                return 0

            lax.fori_loop(0, nv, vec_body, 0)
            return 0

        lax.fori_loop(0, NGRP * NLANE, region_body, 0)

        for k in range(128 // NLANE):
            if k < NGRP2:
                mbuf[pl.ds(k * NLANE, NLANE)] = (
                    posb[pl.ds(k * NLANE, NLANE)] - pos20[k])
            else:
                mbuf[pl.ds(k * NLANE, NLANE)] = zi
        pltpu.sync_copy(mbuf, mc_hbm.at[wid])
        pltpu.sync_copy(s2_v, s2_hbm.at[wid])
        pltpu.sync_copy(d2_v, d2_hbm.at[wid])

    passb(srev_v, drev_v, pos[0:4], mrev_hbm, srev_hbm, drev_hbm)
    passb(smt_v, dmt_v, pos[4:8], mmt_hbm, smt_hbm, dmt_hbm)

    ones = jnp.ones((NLANE,), jnp.float32)
    hstride = lane * (HOWN + 1)
    zf = jnp.zeros((NLANE,), jnp.float32)
    HGRP = NLANE * (HOWN + 1) // NLANE

    def half_hist(dl_v, half, inv_hbm):
        def zero_h(k, _):
            hist_v[pl.ds(k * NLANE, NLANE)] = zf
            return 0

        lax.fori_loop(0, HGRP, zero_h, 0)

        for sg in range(2):
            hbase = (sg * 2 + half) * (NLANE * CAPL)

            def hist_body(g, _, hbase=hbase):
                dv = dl_v[pl.ds(hbase + g * NLANE, NLANE)]
                plsc.addupdate_scatter(hist_v, [hstride + dv], ones)
                return 0

            lax.fori_loop(0, NLANE * CAPL // NLANE, hist_body, 0)

        def inv_body(k, _):
            c16 = jnp.zeros((NLANE,), jnp.float32)
            for l in range(NLANE):
                c16 = c16 + hist_v[pl.ds(l * (HOWN + 1) + k * NLANE, NLANE)]
            inv_v[pl.ds(k * NLANE, NLANE)] = 1.0 / jnp.maximum(c16, 1.0)
            return 0

        lax.fori_loop(0, HOWN // NLANE, inv_body, 0)
        pltpu.sync_copy(inv_v, inv_hbm.at[pl.ds(lo + half * HOWN, HOWN)])

    half_hist(drev_v, 0, irev_hbm)
    half_hist(drev_v, 1, irev_hbm)
    half_hist(dmt_v, 0, imt_hbm)
    half_hist(dmt_v, 1, imt_hbm)


HH = H // 2


@functools.partial(
    pl.kernel,
    out_type=jax.ShapeDtypeStruct((2, NPAD, HH), jnp.float32),
    mesh=_mesh,
    compiler_params=_sc_params,
    scratch_types=[
        pltpu.VMEM((LALLOC,), jnp.int32),
        pltpu.VMEM((LALLOC,), jnp.int32),
        pltpu.VMEM((144,), jnp.int32),
        pltpu.VMEM((160,), jnp.int32),
        pltpu.VMEM((2 * (HOWN + 1), HH), jnp.float32),
        pltpu.VMEM((GCH, HH), jnp.float32),
        pltpu.VMEM((GCH, HH), jnp.float32),
        pltpu.VMEM((SSTG, HH), jnp.float32),
        pltpu.VMEM_SHARED((SQN, HH), jnp.float32),
        pltpu.SemaphoreType.DMA,
        pltpu.SemaphoreType.DMA,
    ],
)
def _sc_agg(p_hbm, slist_hbm, dloc_hbm, mcnt_hbm, out_hbm,
            slist_v, dloc_v, mbuf, btab, acc,
            st0, st1, bounce, ptab, sem0, sem1):
    wid = _wid()
    sub = lax.axis_index("s")
    stages = ((st0, sem0), (st1, sem1))
    pltpu.sync_copy(mcnt_hbm.at[wid], mbuf.at[pl.ds(0, 128)])
    pltpu.sync_copy(slist_hbm.at[wid], slist_v.at[pl.ds(0, CAP2)])
    pltpu.sync_copy(dloc_hbm.at[wid], dloc_v.at[pl.ds(0, CAP2)])

    zi = jnp.zeros((NLANE,), jnp.int32)
    dumpv = jnp.full((NLANE,), DUMP, jnp.int32)
    for k in range((LALLOC - CAP2) // NLANE):
        sl = pl.ds(NULLB + k * NLANE, NLANE)
        slist_v[sl] = zi
        dloc_v[sl] = dumpv

    iota = lax.iota(jnp.int32, NLANE)
    zf = jnp.zeros((NLANE,), jnp.float32)
    nullv = jnp.full((NLANE,), NULLB, jnp.int32)

    def fire(slot, stage, sem):
        base = pl.multiple_of(btab[pl.ds(slot, NLANE)][0], 8)
        idx = slist_v.at[pl.ds(base, GCH)]
        pltpu.async_copy(ptab.at[idx], stage, sem)

    def proc(slot, hoff, stage, sem):
        base = pl.multiple_of(btab[pl.ds(slot, NLANE)][0], 8)
        idx = slist_v.at[pl.ds(base, GCH)]
        pltpu.make_async_copy(ptab.at[idx], stage, sem).wait()

        for eg in range(GCH // NLANE):
            dv = dloc_v[pl.ds(base + eg * NLANE, NLANE)]
            for el in range(NLANE):
                d = dv[el] + hoff
                e = eg * NLANE + el
                for j in range(HH // NLANE):
                    sl = pl.ds(j * NLANE, NLANE)
                    plsc.addupdate(acc.at[d, sl], stage[e, sl])

    def cw_body(cw, _):
        def zero_row(r, _):
            for j in range(HH // NLANE):
                acc[r, pl.ds(j * NLANE, NLANE)] = zf
            return 0

        lax.fori_loop(0, 2 * (HOWN + 1), zero_row, 0)

        def sq_body(sq, _):
            roff = sq * SQN + sub * SSTG
            pltpu.sync_copy(p_hbm.at[cw, pl.ds(roff, SSTG)],
                            bounce.at[pl.ds(0, SSTG)])
            pltpu.sync_copy(bounce.at[pl.ds(0, SSTG)],
                            ptab.at[pl.ds(sub * SSTG, SSTG)])
            plsc.subcore_barrier()

            def h_body(h, _):
                g = sq * 2 + h
                hoff = h * (HOWN + 1)

                def build(r, cum):
                    mr = mbuf[pl.ds(NLANE * g + r, NLANE)][0]
                    trips = (mr + (GCH - 1)) // GCH
                    bases = (g * NLANE + r) * CAPL2 + GCH * iota
                    plsc.store_scatter(btab, [cum + iota], bases,
                                       mask=iota < trips)
                    return cum + trips

                T = lax.fori_loop(0, NLANE, build, 0)
                plsc.store_scatter(btab, [T + iota], nullv, mask=iota < RING)
                tpad = (T + (RING - 1)) // RING

                @pl.when(T > 0)
                def _():
                    for k in range(RING):
                        fire(k, *stages[k])

                def ring_body(gg, _):
                    for k in range(RING):
                        slot = RING * gg + k
                        proc(slot, hoff, *stages[k])

                        @pl.when(slot + RING < tpad * RING)
                        def _():
                            fire(slot + RING, *stages[k])
                    return 0

                lax.fori_loop(0, tpad, ring_body, 0)
                return 0

            lax.fori_loop(0, 2, h_body, 0)
            plsc.subcore_barrier()
            return 0

        lax.fori_loop(0, 4, sq_body, 0)
        pltpu.sync_copy(
            acc.at[pl.ds(0, HOWN)],
            out_hbm.at[cw, pl.ds(wid * OWN, HOWN)])
        pltpu.sync_copy(
            acc.at[pl.ds(HOWN + 1, HOWN)],
            out_hbm.at[cw, pl.ds(wid * OWN + HOWN, HOWN)])
        return 0

    lax.fori_loop(0, 2, cw_body, 0)


@functools.partial(
    pl.kernel,
    out_type=jax.ShapeDtypeStruct((E, H), jnp.float32),
    mesh=_mesh,
    compiler_params=_sc_params,
    scratch_types=[
        pltpu.VMEM((EPT,), jnp.int32),
        pltpu.VMEM((EPT,), jnp.int32),
        pltpu.VMEM((PCH, H), jnp.float32),
        pltpu.VMEM((PCH, H), jnp.float32),
        pltpu.VMEM((PCH, H), jnp.float32),
        pltpu.VMEM((PCH, H), jnp.float32),
        pltpu.VMEM((PCH, H), jnp.float32),
        pltpu.VMEM((PCH, H), jnp.float32),
        pltpu.VMEM((PCH, H), jnp.float32),
        pltpu.VMEM((PCH, H), jnp.float32),
        pltpu.VMEM((PCH, H), jnp.float32),
        pltpu.SemaphoreType.DMA,
        pltpu.SemaphoreType.DMA,
        pltpu.SemaphoreType.DMA,
        pltpu.SemaphoreType.DMA,
        pltpu.SemaphoreType.DMA,
        pltpu.SemaphoreType.DMA,
        pltpu.SemaphoreType.DMA,
        pltpu.SemaphoreType.DMA,
        pltpu.SemaphoreType.DMA,
    ],
)
def _sc_pairgather(a_hbm, b_hbm, e0_hbm, e1_hbm, out_hbm,
                   i0_v, i1_v, a0, a1, a2, b0, b1, b2, r0, r1, r2,
                   sa0, sa1, sa2, sb0, sb1, sb2, sw0, sw1, sw2):
    wid = _wid()
    lo = wid * EPT
    pltpu.sync_copy(e0_hbm.at[pl.ds(lo, EPT)], i0_v)
    pltpu.sync_copy(e1_hbm.at[pl.ds(lo, EPT)], i1_v)
    stages = ((a0, b0, r0, sa0, sb0, sw0),
              (a1, b1, r1, sa1, sb1, sw1),
              (a2, b2, r2, sa2, sb2, sw2))

    def fire(c, av, bv, sa, sb):
        off = pl.multiple_of(c * PCH, 8)
        pltpu.async_copy(a_hbm.at[i0_v.at[pl.ds(off, PCH)]], av, sa)
        pltpu.async_copy(b_hbm.at[i1_v.at[pl.ds(off, PCH)]], bv, sb)

    for k in range(PRING):
        fire(k, stages[k][0], stages[k][1], stages[k][3], stages[k][4])

    def proc(c, av, bv, rv, sa, sb, sw):
        off = pl.multiple_of(c * PCH, 8)
        pltpu.make_async_copy(a_hbm.at[i0_v.at[pl.ds(off, PCH)]], av, sa).wait()
        pltpu.make_async_copy(b_hbm.at[i1_v.at[pl.ds(off, PCH)]], bv, sb).wait()

        @pl.when(c >= PRING)
        def _():
            pltpu.make_async_copy(
                rv, out_hbm.at[pl.ds(lo, PCH)], sw).wait()

        def row_body(e, _):
            for j in range(H // NLANE):
                sl = pl.ds(j * NLANE, NLANE)
                rv[e, sl] = av[e, sl] + bv[e, sl]
            return 0

        lax.fori_loop(0, PCH, row_body, 0)
        pltpu.async_copy(rv, out_hbm.at[pl.ds(lo + c * PCH, PCH)], sw)

        @pl.when(c + PRING < NCHUNK)
        def _():
            fire(c + PRING, av, bv, sa, sb)

    def loop_body(c, _):
        m = lax.rem(c, PRING)
        for k in range(PRING):
            @pl.when(m == k)
            def _():
                proc(c, *stages[k])
        return 0

    lax.fori_loop(0, NCHUNK, loop_body, 0)
    for k in range(PRING):
        pltpu.make_async_copy(
            stages[k][2], out_hbm.at[pl.ds(lo, PCH)], stages[k][5]).wait()


_BLK = 1000


def _tc_init_disease(disease_x, lin_W, lin_b, disease_emb):
    def body(dx, w, b, emb, o):
        o[...] = jnp.dot(dx[...], w[...],
                         preferred_element_type=jnp.float32) + b[...] + emb[...]

    return pl.pallas_call(
        body,
        grid=(N // _BLK,),
        in_specs=[
            pl.BlockSpec((_BLK, 10), lambda i: (i, 0)),
            pl.BlockSpec((10, H), lambda i: (0, 0)),
            pl.BlockSpec((1, H), lambda i: (0, 0)),
            pl.BlockSpec((_BLK, H), lambda i: (i, 0)),
        ],
        out_specs=pl.BlockSpec((_BLK, H), lambda i: (i, 0)),
        out_shape=jax.ShapeDtypeStruct((N, H), jnp.float32),
    )(disease_x, lin_W, lin_b, disease_emb)


def _tc_layer_mats(xd, xs, wl_rev, wr_rev, wl_mt, wr_mt):
    def body(xd_r, xs_r, a, b, c, d, p_rev, s_d, p_mt, s_s):
        xdv = xd_r[...]
        xsv = xs_r[...]
        pr = jnp.dot(xsv, a[...], preferred_element_type=jnp.float32)
        p_rev[0] = pr[:, :HH]
        p_rev[1] = pr[:, HH:]
        s_d[...] = jnp.dot(xdv, b[...], preferred_element_type=jnp.float32)
        pm = jnp.dot(xdv, c[...], preferred_element_type=jnp.float32)
        p_mt[0] = pm[:, :HH]
        p_mt[1] = pm[:, HH:]
        s_s[...] = jnp.dot(xsv, d[...], preferred_element_type=jnp.float32)

    full = pl.BlockSpec((H, H), lambda i: (0, 0))
    rows = pl.BlockSpec((_BLK, H), lambda i: (i, 0))
    halves = pl.BlockSpec((2, _BLK, HH), lambda i: (0, i, 0))
    shp = jax.ShapeDtypeStruct((N, H), jnp.float32)
    shp2 = jax.ShapeDtypeStruct((2, 4 * SQN, HH), jnp.float32)
    return pl.pallas_call(
        body,
        grid=(N // _BLK,),
        in_specs=[rows, rows, full, full, full, full],
        out_specs=[halves, rows, halves, rows],
        out_shape=[shp2, shp, shp2, shp],
    )(xd, xs, wl_rev, wr_rev, wl_mt, wr_mt)


def _tc_combine(aggd, invd, sd, bld, aggs, invs, ss, bls, relu):
    def body(ad, idv, sdv, bd, as_, isv, ssv, bs, xd_o, xs_o):
        iv = idv[...]
        nd = jnp.concatenate([ad[0], ad[1]], axis=1) * iv + sdv[...] + bd[...]
        iv2 = isv[...]
        ns = jnp.concatenate([as_[0], as_[1]], axis=1) * iv2 + ssv[...] + bs[...]
        if relu:
            nd = jnp.maximum(nd, 0.0)
            ns = jnp.maximum(ns, 0.0)
        xd_o[...] = nd
        xs_o[...] = ns

    rows = pl.BlockSpec((_BLK, H), lambda i: (i, 0))
    halves = pl.BlockSpec((2, _BLK, HH), lambda i: (0, i, 0))
    col = pl.BlockSpec((_BLK, 1), lambda i: (i, 0))
    bias = pl.BlockSpec((1, H), lambda i: (0, 0))
    shp = jax.ShapeDtypeStruct((N, H), jnp.float32)
    return pl.pallas_call(
        body,
        grid=(N // _BLK,),
        in_specs=[halves, col, rows, bias, halves, col, rows, bias],
        out_specs=[rows, rows],
        out_shape=[shp, shp],
    )(aggd, invd, sd, bld, aggs, invs, ss, bls)


def _tc_mlp_head(xd, xs, w_top, w_bot, b1):
    def body(xd_r, xs_r, wt, wb, b, a_o, b_o):
        a_o[...] = jnp.dot(xd_r[...], wt[...],
                           preferred_element_type=jnp.float32)
        b_o[...] = jnp.dot(xs_r[...], wb[...],
                           preferred_element_type=jnp.float32) + b[...]

    rows = pl.BlockSpec((_BLK, H), lambda i: (i, 0))
    full = pl.BlockSpec((H, H), lambda i: (0, 0))
    shp = jax.ShapeDtypeStruct((N, H), jnp.float32)
    return pl.pallas_call(
        body,
        grid=(N // _BLK,),
        in_specs=[rows, rows, full, full, pl.BlockSpec((1, H), lambda i: (0, 0))],
        out_specs=[rows, rows],
        out_shape=[shp, shp],
    )(xd, xs, w_top, w_bot, b1)


_MBLK = 1000


def _tc_mlp(h1, w2, b2, w3, b3, w4, b4):
    def body(h_r, w2r, b2r, w3r, b3r, w4r, b4r, o):
        h = jnp.maximum(h_r[...], 0.0)
        h = jnp.maximum(jnp.dot(h, w2r[...],
                                preferred_element_type=jnp.float32) + b2r[...], 0.0)
        h = jnp.maximum(jnp.dot(h, w3r[...],
                                preferred_element_type=jnp.float32) + b3r[...], 0.0)
        o[...] = jnp.dot(h, w4r[...],
                         preferred_element_type=jnp.float32) + b4r[...]

    return pl.pallas_call(
        body,
        grid=(E // _MBLK,),
        in_specs=[
            pl.BlockSpec((_MBLK, H), lambda i: (i, 0)),
            pl.BlockSpec((H, 128), lambda i: (0, 0)),
            pl.BlockSpec((1, 128), lambda i: (0, 0)),
            pl.BlockSpec((128, 64), lambda i: (0, 0)),
            pl.BlockSpec((1, 64), lambda i: (0, 0)),
            pl.BlockSpec((64, 1), lambda i: (0, 0)),
            pl.BlockSpec((1, 1), lambda i: (0, 0)),
        ],
        out_specs=pl.BlockSpec((_MBLK, 1), lambda i: (i, 0)),
        out_shape=jax.ShapeDtypeStruct((E, 1), jnp.float32),
    )(h1, w2, b2, w3, b3, w4, b4)


def kernel(drug_node_id, disease_x, disease_node_id, edge_index,
           edge_label_index, params):
    xd = params["drug_emb"]
    xs = _tc_init_disease(disease_x, params["lin_W"],
                          params["lin_b"].reshape(1, H), params["disease_emb"])

    (sl_rev, dl_rev, mc_rev, inv_rev,
     sl_mt, dl_mt, mc_mt, inv_mt) = _sc_prep(edge_index[0], edge_index[1])
    invd = inv_rev[:N].reshape(N, 1)
    invs = inv_mt[:N].reshape(N, 1)

    for i in range(4):
        lp = params["convs"][i]
        p_rev, s_d, p_mt, s_s = _tc_layer_mats(
            xd, xs, lp["rev"]["Wl"], lp["rev"]["Wr"],
            lp["mt"]["Wl"], lp["mt"]["Wr"])
        agg_d = _sc_agg(p_rev, sl_rev, dl_rev, mc_rev)
        agg_s = _sc_agg(p_mt, sl_mt, dl_mt, mc_mt)
        xd, xs = _tc_combine(
            agg_d[:, :N], invd, s_d, lp["rev"]["bl"].reshape(1, H),
            agg_s[:, :N], invs, s_s, lp["mt"]["bl"].reshape(1, H),
            relu=(i < 3))

    w1, b1 = params["fc"][0]
    a_tab, b_tab = _tc_mlp_head(xd, xs, w1[:H], w1[H:], b1.reshape(1, H))
    h1 = _sc_pairgather(a_tab, b_tab, edge_label_index[0], edge_label_index[1])

    w2, b2 = params["fc"][1]
    w3, b3 = params["fc"][2]
    w4, b4 = params["fc"][3]
    out = _tc_mlp(h1, w2, b2.reshape(1, 128), w3, b3.reshape(1, 64),
                  w4, b4.reshape(1, 1))
    return jnp.squeeze(out, -1)
